# baseline (speedup 1.0000x reference)
import jax
import jax.numpy as jnp
from jax import lax
from jax.experimental import pallas as pl
from jax.experimental.pallas import tpu as pltpu
from jax.experimental.pallas import tpu_sc as plsc

NUM_TABLE_ROWS = 100000
DIM = 128
BATCH = 4096 * 50
NUM_WORKERS = 32
PER_WORKER = BATCH // NUM_WORKERS
CHUNK = 128
N_CHUNKS = PER_WORKER // CHUNK
NBUF = 7


def _emb_kernel(ids_hbm, table_hbm, out_hbm, idx_v, *bufs):
    rows = list(bufs[:NBUF])
    gsem = list(bufs[NBUF:2 * NBUF])
    ssem = list(bufs[2 * NBUF:])
    wid = lax.axis_index("s") * 2 + lax.axis_index("c")
    base = wid * PER_WORKER
    pltpu.sync_copy(ids_hbm.at[pl.ds(base, PER_WORKER)], idx_v)

    def g_copy(c, b):
        idx = idx_v.at[pl.ds(c * CHUNK, CHUNK)]
        return pltpu.make_async_copy(table_hbm.at[idx], rows[b], gsem[b])

    def s_copy(c, b):
        dst = out_hbm.at[pl.ds(base + c * CHUNK, CHUNK)]
        return pltpu.make_async_copy(rows[b], dst, ssem[b])

    for c in range(min(NBUF - 1, N_CHUNKS)):
        g_copy(c, c).start()

    for c in range(N_CHUNKS):
        b = c % NBUF
        pc = c + NBUF - 1
        if pc < N_CHUNKS:
            pb = pc % NBUF
            if pc - NBUF >= 0:
                s_copy(pc - NBUF, pb).wait()
            g_copy(pc, pb).start()
        g_copy(c, b).wait()
        s_copy(c, b).start()

    for c in range(max(0, N_CHUNKS - NBUF), N_CHUNKS):
        s_copy(c, c % NBUF).wait()


@jax.jit
def _lookup(ids_flat, embeddings):
    mesh = plsc.VectorSubcoreMesh(core_axis_name="c", subcore_axis_name="s")
    return pl.kernel(
        _emb_kernel,
        out_type=jax.ShapeDtypeStruct((BATCH, DIM), jnp.float32),
        mesh=mesh,
        scratch_types=(
            [pltpu.VMEM((PER_WORKER,), jnp.int32)]
            + [pltpu.VMEM((CHUNK, DIM), jnp.float32)] * NBUF
            + [pltpu.SemaphoreType.DMA] * (2 * NBUF)
        ),
    )(ids_flat, embeddings)


def kernel(token_ids, embeddings):
    b, t = token_ids.shape
    out = _lookup(token_ids.reshape(-1), embeddings)
    return out.reshape(b, t, DIM)


# speedup vs baseline: 1.0054x; 1.0054x over previous
import jax
import jax.numpy as jnp
from jax import lax
from jax.experimental import pallas as pl
from jax.experimental.pallas import tpu as pltpu
from jax.experimental.pallas import tpu_sc as plsc

NUM_TABLE_ROWS = 100000
DIM = 128
BATCH = 4096 * 50
NUM_WORKERS = 32
PER_WORKER = BATCH // NUM_WORKERS
CHUNK = 128
G_PER = 2
SUPER = CHUNK * G_PER
N_SUP = PER_WORKER // SUPER
NBUF = 3


def _emb_kernel(ids_hbm, table_hbm, out_hbm, idx_v, *bufs):
    rows = list(bufs[:NBUF])
    gsem = list(bufs[NBUF:NBUF + NBUF * G_PER])
    ssem = list(bufs[NBUF + NBUF * G_PER:])
    wid = lax.axis_index("s") * 2 + lax.axis_index("c")
    base = wid * PER_WORKER
    pltpu.sync_copy(ids_hbm.at[pl.ds(base, PER_WORKER)], idx_v)

    def g_copy(s, h, b):
        idx = idx_v.at[pl.ds((s * G_PER + h) * CHUNK, CHUNK)]
        return pltpu.make_async_copy(
            table_hbm.at[idx], rows[b].at[pl.ds(h * CHUNK, CHUNK)],
            gsem[b * G_PER + h])

    def s_copy(s, b):
        dst = out_hbm.at[pl.ds(base + s * SUPER, SUPER)]
        return pltpu.make_async_copy(rows[b], dst, ssem[b])

    for s in range(min(NBUF - 1, N_SUP)):
        for h in range(G_PER):
            g_copy(s, h, s).start()

    for s in range(N_SUP):
        b = s % NBUF
        ps = s + NBUF - 1
        if ps < N_SUP:
            pb = ps % NBUF
            if ps - NBUF >= 0:
                s_copy(ps - NBUF, pb).wait()
            for h in range(G_PER):
                g_copy(ps, h, pb).start()
        for h in range(G_PER):
            g_copy(s, h, b).wait()
        s_copy(s, b).start()

    for s in range(max(0, N_SUP - NBUF), N_SUP):
        s_copy(s, s % NBUF).wait()


@jax.jit
def _lookup(ids_flat, embeddings):
    mesh = plsc.VectorSubcoreMesh(core_axis_name="c", subcore_axis_name="s")
    return pl.kernel(
        _emb_kernel,
        out_type=jax.ShapeDtypeStruct((BATCH, DIM), jnp.float32),
        mesh=mesh,
        scratch_types=(
            [pltpu.VMEM((PER_WORKER,), jnp.int32)]
            + [pltpu.VMEM((SUPER, DIM), jnp.float32)] * NBUF
            + [pltpu.SemaphoreType.DMA] * (NBUF * G_PER + NBUF)
        ),
    )(ids_flat, embeddings)


def kernel(token_ids, embeddings):
    b, t = token_ids.shape
    out = _lookup(token_ids.reshape(-1), embeddings)
    return out.reshape(b, t, DIM)
